# baseline (speedup 1.0000x reference)
import functools

import jax
import jax.numpy as jnp
from jax import lax
from jax.experimental import pallas as pl
from jax.experimental.pallas import tpu as pltpu
from jax.experimental.pallas import tpu_sc as plsc

_N = 1000
_E = 64000
_NPAD = 1024
_NC = 2
_NS = 16
_NW = _NC * _NS
_EPW = _E // _NW
_L = 16
_TW = 512


_sc_mesh = plsc.VectorSubcoreMesh(core_axis_name="c", subcore_axis_name="s")


@functools.partial(
    pl.kernel,
    mesh=_sc_mesh,
    compiler_params=pltpu.CompilerParams(needs_layout_passes=False),
    out_type=jax.ShapeDtypeStruct((_NW, 1, 2 * _NPAD), jnp.float32),
    scratch_types=[
        pltpu.VMEM((2, _TW), jnp.int32),
        pltpu.VMEM((2, _TW), jnp.int32),
        pltpu.VMEM((_N,), jnp.float32),
        pltpu.VMEM((2 * _NPAD,), jnp.float32),
    ],
)
def _sc_seg(edge_hbm, x_hbm, out_hbm, e2a_v, e2b_v, x_v, cnt_v):
    wid = lax.axis_index("s") * _NC + lax.axis_index("c")
    n_t = jnp.where(wid < 29, 4, 3)
    base_t = jnp.where(wid < 29, 4 * wid, 116 + 3 * (wid - 29))
    pltpu.sync_copy(x_hbm, x_v)

    @plsc.parallel_loop(0, 2 * _NPAD // _L, unroll=4)
    def _zero(i):
        cnt_v[pl.ds(i * _L, _L)] = jnp.zeros((_L,), jnp.float32)

    ones = jnp.full((_L,), 1.0, jnp.float32)
    zidx = jnp.zeros((_L,), jnp.int32)
    oidx = jnp.ones((_L,), jnp.int32)
    lane = lax.iota(jnp.int32, _L)
    bufs = (e2a_v, e2b_v)

    for j in range(4):
        buf = bufs[j % 2]

        @pl.when(j < n_t)
        def _chunk():
            pltpu.sync_copy(
                edge_hbm.at[:, pl.ds((base_t + j) * _TW, _TW)], buf)

            @plsc.parallel_loop(0, _TW // _L, unroll=4)
            def _edges(i):
                c = i * _L + lane
                row = plsc.load_gather(buf, [zidx, c])
                col = plsc.load_gather(buf, [oidx, c])
                xv = plsc.load_gather(x_v, [col])
                off = jnp.where(xv > 0.999, 0, _NPAD)
                plsc.addupdate_scatter(cnt_v, [row + off], ones)

    pltpu.sync_copy(cnt_v, out_hbm.at[wid, 0])


def _tc_loss_fn(part_ref, x_ref, y_ref, o_ref, loss_ref):
    s = jnp.sum(part_ref[...], axis=0)
    n1 = s[:, :_N]
    n0 = s[:, _NPAD:_NPAD + _N]
    xr = x_ref[...]
    a_row = n1
    b_row = xr * 131072.0 + n0

    ii = lax.broadcasted_iota(jnp.int32, (_N, _N), 0)
    jj = lax.broadcasted_iota(jnp.int32, (_N, _N), 1)
    ident = (ii == jj).astype(jnp.float32)
    dn = (((1,), (1,)), ((), ()))
    a_col = lax.dot_general(ident, a_row, dn,
                            preferred_element_type=jnp.float32)
    b_col = lax.dot_general(ident, b_row, dn,
                            preferred_element_type=jnp.float32)
    eq = (a_col == a_row) & (b_col == b_row)
    counts = jnp.sum(eq.astype(jnp.float32), axis=1, keepdims=True)
    w = lax.rsqrt(counts)

    o = o_ref[...]
    l0 = o[:, 0:1]
    l1 = o[:, 1:2]
    m = jnp.maximum(l0, l1)
    lse = m + jnp.log(jnp.exp(l0 - m) + jnp.exp(l1 - m))
    ly = jnp.where(y_ref[...] == 0, l0, l1)
    node_loss = lse - ly
    num = jnp.sum(node_loss * w)
    den = jnp.sum(w)
    loss_ref[...] = (num / den)[None, None]


def kernel(out, x, y, edge_index):
    partials = _sc_seg(edge_index, x)
    xr = x.reshape(1, _N)
    yc = y.reshape(_N, 1)
    res = pl.pallas_call(
        _tc_loss_fn,
        out_shape=jax.ShapeDtypeStruct((1, 1), jnp.float32),
    )(partials, xr, yc, out)
    return res[0, 0]


# speedup vs baseline: 1.0129x; 1.0129x over previous
import functools

import jax
import jax.numpy as jnp
from jax import lax
from jax.experimental import pallas as pl
from jax.experimental.pallas import tpu as pltpu
from jax.experimental.pallas import tpu_sc as plsc

_N = 1000
_E = 64000
_NPAD = 1024
_NC = 2
_NS = 16
_NW = _NC * _NS
_EPW = _E // _NW
_L = 16
_TW = 512


_sc_mesh = plsc.VectorSubcoreMesh(core_axis_name="c", subcore_axis_name="s")


@functools.partial(
    pl.kernel,
    mesh=_sc_mesh,
    compiler_params=pltpu.CompilerParams(needs_layout_passes=False),
    out_type=jax.ShapeDtypeStruct((_NW, 1, 2 * _NPAD), jnp.float32),
    scratch_types=[
        pltpu.VMEM((2, _TW), jnp.int32),
        pltpu.VMEM((2, _TW), jnp.int32),
        pltpu.VMEM((_N,), jnp.float32),
        pltpu.VMEM((2 * _NPAD,), jnp.float32),
        pltpu.SemaphoreType.DMA,
        pltpu.SemaphoreType.DMA,
    ],
)
def _sc_seg(edge_hbm, x_hbm, out_hbm, e2a_v, e2b_v, x_v, cnt_v, sem0, sem1):
    wid = lax.axis_index("s") * _NC + lax.axis_index("c")
    n_t = jnp.where(wid < 29, 4, 3)
    base_t = jnp.where(wid < 29, 4 * wid, 116 + 3 * (wid - 29))
    pltpu.sync_copy(x_hbm, x_v)

    @plsc.parallel_loop(0, 2 * _NPAD // _L, unroll=4)
    def _zero(i):
        cnt_v[pl.ds(i * _L, _L)] = jnp.zeros((_L,), jnp.float32)

    ones = jnp.full((_L,), 1.0, jnp.float32)
    zidx = jnp.zeros((_L,), jnp.int32)
    oidx = jnp.ones((_L,), jnp.int32)
    lane = lax.iota(jnp.int32, _L)
    bufs = (e2a_v, e2b_v)
    sems = (sem0, sem1)

    def compute(buf):
        @plsc.parallel_loop(0, _TW // _L, unroll=4)
        def _edges(i):
            c = i * _L + lane
            row = plsc.load_gather(buf, [zidx, c])
            col = plsc.load_gather(buf, [oidx, c])
            xv = plsc.load_gather(x_v, [col])
            off = jnp.where(xv > 0.999, 0, _NPAD)
            plsc.addupdate_scatter(cnt_v, [row + off], ones)

    def start(j):
        t = jnp.minimum(base_t + j, 124)
        return pltpu.async_copy(
            edge_hbm.at[:, pl.ds(t * _TW, _TW)], bufs[j % 2], sems[j % 2])

    cps = [start(0)]
    for j in range(4):
        cps[j].wait()
        if j + 1 < 4:
            cps.append(start(j + 1))
        if j < 3:
            compute(bufs[j % 2])
        else:
            @pl.when(wid < 29)
            def _last():
                compute(bufs[j % 2])

    pltpu.sync_copy(cnt_v, out_hbm.at[wid, 0])


def _tc_loss_fn(part_ref, x_ref, y_ref, o_ref, loss_ref):
    s = jnp.sum(part_ref[...], axis=0)
    n1 = s[:, :_N]
    n0 = s[:, _NPAD:_NPAD + _N]
    xr = x_ref[...]
    a_row = n1
    b_row = xr * 131072.0 + n0

    ii = lax.broadcasted_iota(jnp.int32, (_N, _N), 0)
    jj = lax.broadcasted_iota(jnp.int32, (_N, _N), 1)
    ident = (ii == jj).astype(jnp.float32)
    dn = (((1,), (1,)), ((), ()))
    a_col = lax.dot_general(ident, a_row, dn,
                            preferred_element_type=jnp.float32)
    b_col = lax.dot_general(ident, b_row, dn,
                            preferred_element_type=jnp.float32)
    eq = (a_col == a_row) & (b_col == b_row)
    counts = jnp.sum(eq.astype(jnp.float32), axis=1, keepdims=True)
    w = lax.rsqrt(counts)

    o = o_ref[...]
    l0 = o[:, 0:1]
    l1 = o[:, 1:2]
    m = jnp.maximum(l0, l1)
    lse = m + jnp.log(jnp.exp(l0 - m) + jnp.exp(l1 - m))
    ly = jnp.where(y_ref[...] == 0, l0, l1)
    node_loss = lse - ly
    num = jnp.sum(node_loss * w)
    den = jnp.sum(w)
    loss_ref[...] = (num / den)[None, None]


def kernel(out, x, y, edge_index):
    partials = _sc_seg(edge_index, x)
    xr = x.reshape(1, _N)
    yc = y.reshape(_N, 1)
    res = pl.pallas_call(
        _tc_loss_fn,
        out_shape=jax.ShapeDtypeStruct((1, 1), jnp.float32),
    )(partials, xr, yc, out)
    return res[0, 0]


# speedup vs baseline: 1.0885x; 1.0747x over previous
import functools

import jax
import jax.numpy as jnp
from jax import lax
from jax.experimental import pallas as pl
from jax.experimental.pallas import tpu as pltpu
from jax.experimental.pallas import tpu_sc as plsc

_N = 1000
_E = 64000
_NPAD = 1024
_NC = 2
_NS = 16
_NW = _NC * _NS
_EPW = _E // _NW
_L = 16


_sc_mesh = plsc.VectorSubcoreMesh(core_axis_name="c", subcore_axis_name="s")


@functools.partial(
    pl.kernel,
    mesh=_sc_mesh,
    compiler_params=pltpu.CompilerParams(needs_layout_passes=False),
    out_type=jax.ShapeDtypeStruct((_NW, 1, 2 * _NPAD), jnp.float32),
    scratch_types=[
        pltpu.VMEM((_EPW,), jnp.int32),
        pltpu.VMEM((_EPW,), jnp.int32),
        pltpu.VMEM((_N,), jnp.float32),
        pltpu.VMEM((2 * _NPAD,), jnp.float32),
        pltpu.SemaphoreType.DMA,
    ],
)
def _sc_seg(edge_hbm, x_hbm, out_hbm, row_v, col_v, x_v, cnt_v, sem):
    wid = lax.axis_index("s") * _NC + lax.axis_index("c")
    base = wid * _EPW
    cp_r = pltpu.async_copy(edge_hbm.at[pl.ds(base, _EPW)], row_v, sem)
    cp_c = pltpu.async_copy(edge_hbm.at[pl.ds(_E + base, _EPW)], col_v, sem)
    cp_x = pltpu.async_copy(x_hbm, x_v, sem)

    @plsc.parallel_loop(0, 2 * _NPAD // _L, unroll=8)
    def _zero(i):
        cnt_v[pl.ds(i * _L, _L)] = jnp.zeros((_L,), jnp.float32)

    cp_r.wait()
    cp_c.wait()
    cp_x.wait()

    ones = jnp.full((_L,), 1.0, jnp.float32)

    @plsc.parallel_loop(0, _EPW // _L, unroll=5)
    def _edges(i):
        b = i * _L
        col = col_v[pl.ds(b, _L)]
        row = row_v[pl.ds(b, _L)]
        xv = plsc.load_gather(x_v, [col])
        off = jnp.where(xv > 0.999, 0, _NPAD)
        plsc.addupdate_scatter(cnt_v, [row + off], ones)

    pltpu.sync_copy(cnt_v, out_hbm.at[wid, 0])


def _tc_loss_fn(part_ref, x_ref, y_ref, o_ref, loss_ref):
    s = jnp.sum(part_ref[...], axis=0)
    n1 = s[:, :_N]
    n0 = s[:, _NPAD:_NPAD + _N]
    xr = x_ref[...]
    a_row = n1
    b_row = xr * 131072.0 + n0

    ii = lax.broadcasted_iota(jnp.int32, (_N, _N), 0)
    jj = lax.broadcasted_iota(jnp.int32, (_N, _N), 1)
    ident = (ii == jj).astype(jnp.float32)
    dn = (((1,), (1,)), ((), ()))
    a_col = lax.dot_general(ident, a_row, dn,
                            preferred_element_type=jnp.float32)
    b_col = lax.dot_general(ident, b_row, dn,
                            preferred_element_type=jnp.float32)
    eq = (a_col == a_row) & (b_col == b_row)
    counts = jnp.sum(eq.astype(jnp.float32), axis=1, keepdims=True)
    w = lax.rsqrt(counts)

    o = o_ref[...]
    l0 = o[:, 0:1]
    l1 = o[:, 1:2]
    m = jnp.maximum(l0, l1)
    lse = m + jnp.log(jnp.exp(l0 - m) + jnp.exp(l1 - m))
    ly = jnp.where(y_ref[...] == 0, l0, l1)
    node_loss = lse - ly
    num = jnp.sum(node_loss * w)
    den = jnp.sum(w)
    loss_ref[...] = (num / den)[None, None]


def kernel(out, x, y, edge_index):
    partials = _sc_seg(edge_index.reshape(2 * _E), x)
    xr = x.reshape(1, _N)
    yc = y.reshape(_N, 1)
    res = pl.pallas_call(
        _tc_loss_fn,
        out_shape=jax.ShapeDtypeStruct((1, 1), jnp.float32),
    )(partials, xr, yc, out)
    return res[0, 0]
